# baseline (speedup 1.0000x reference)
import functools

import jax
import jax.numpy as jnp
from jax import lax
from jax.experimental import pallas as pl
from jax.experimental.pallas import tpu as pltpu
from jax.experimental.pallas import tpu_sc as plsc


def kernel(tokens, W_pos):
    batch, seq_len = tokens.shape
    d_model = W_pos.shape[1]

    info = plsc.get_sparse_core_info()
    NC, NS = info.num_cores, info.num_subcores
    NW = NC * NS
    rows_per_w = seq_len // NW
    C = 16
    n_chunks = rows_per_w // C

    mesh = plsc.VectorSubcoreMesh(core_axis_name="c", subcore_axis_name="s")

    @functools.partial(
        pl.kernel,
        mesh=mesh,
        out_type=jax.ShapeDtypeStruct((batch, seq_len, d_model), W_pos.dtype),
        scratch_types=[
            pltpu.VMEM((C, d_model), jnp.float32),
            pltpu.VMEM((C, d_model), jnp.float32),
            pltpu.SemaphoreType.DMA,
            pltpu.SemaphoreType.DMA,
            pltpu.SemaphoreType.DMA,
            pltpu.SemaphoreType.DMA,
        ],
    )
    def sc_broadcast(w_hbm, out_hbm, buf0, buf1, rsem0, rsem1, wsem0, wsem1):
        wid = lax.axis_index("s") * NC + lax.axis_index("c")
        base0 = wid * rows_per_w
        bufs = (buf0, buf1)
        rsems = (rsem0, rsem1)
        wsems = (wsem0, wsem1)

        write_handles = [None, None]
        read_handle = [None, None]
        read_handle[0] = pltpu.async_copy(
            w_hbm.at[pl.ds(base0, C)], bufs[0], rsems[0])
        for c in range(n_chunks):
            s = c & 1
            if c + 1 < n_chunks:
                s2 = 1 - s
                if write_handles[s2] is not None:
                    for h in write_handles[s2]:
                        h.wait()
                read_handle[s2] = pltpu.async_copy(
                    w_hbm.at[pl.ds(base0 + (c + 1) * C, C)], bufs[s2], rsems[s2])
            read_handle[s].wait()
            write_handles[s] = [
                pltpu.async_copy(
                    bufs[s], out_hbm.at[b, pl.ds(base0 + c * C, C)], wsems[s])
                for b in range(batch)
            ]
        for s in (0, 1):
            if write_handles[s] is not None:
                for h in write_handles[s]:
                    h.wait()

    return sc_broadcast(W_pos)


# speedup vs baseline: 1.4195x; 1.4195x over previous
import jax
import jax.numpy as jnp
from jax.experimental import pallas as pl
from jax.experimental.pallas import tpu as pltpu


def kernel(tokens, W_pos):
    batch, seq_len = tokens.shape
    d_model = W_pos.shape[1]
    BS = 512
    n_chunks = seq_len // BS

    def body(w_any, o_any, buf0, buf1, rsem0, rsem1, wsem0, wsem1):
        bufs = (buf0, buf1)
        rsems = (rsem0, rsem1)
        wsems = (wsem0, wsem1)
        read_h = [None, None]
        write_h = [None, None]

        read_h[0] = pltpu.make_async_copy(
            w_any.at[pl.ds(0, BS)], bufs[0], rsems[0])
        read_h[0].start()
        for c in range(n_chunks):
            s = c & 1
            if c + 1 < n_chunks:
                s2 = 1 - s
                if write_h[s2] is not None:
                    for h in write_h[s2]:
                        h.wait()
                read_h[s2] = pltpu.make_async_copy(
                    w_any.at[pl.ds((c + 1) * BS, BS)], bufs[s2], rsems[s2])
                read_h[s2].start()
            read_h[s].wait()
            write_h[s] = []
            for b in range(batch):
                h = pltpu.make_async_copy(
                    bufs[s], o_any.at[b, pl.ds(c * BS, BS)], wsems[s])
                h.start()
                write_h[s].append(h)
        for s in (0, 1):
            if write_h[s] is not None:
                for h in write_h[s]:
                    h.wait()

    return pl.pallas_call(
        body,
        in_specs=[pl.BlockSpec(memory_space=pl.ANY)],
        out_specs=pl.BlockSpec(memory_space=pl.ANY),
        out_shape=jax.ShapeDtypeStruct((batch, seq_len, d_model), W_pos.dtype),
        scratch_shapes=[
            pltpu.VMEM((BS, d_model), jnp.float32),
            pltpu.VMEM((BS, d_model), jnp.float32),
            pltpu.SemaphoreType.DMA,
            pltpu.SemaphoreType.DMA,
            pltpu.SemaphoreType.DMA,
            pltpu.SemaphoreType.DMA,
        ],
    )(W_pos)


# speedup vs baseline: 1.4767x; 1.0403x over previous
import jax
import jax.numpy as jnp
from jax.experimental import pallas as pl
from jax.experimental.pallas import tpu as pltpu


def kernel(tokens, W_pos):
    batch, seq_len = tokens.shape
    d_model = W_pos.shape[1]
    BS = 512
    n_chunks = seq_len // BS
    NBUF = 4

    def body(w_any, o_any, *scratch):
        bufs = scratch[:NBUF]
        rsems = scratch[NBUF:2 * NBUF]
        wsems = scratch[2 * NBUF:]
        read_h = [None] * NBUF
        write_h = [None] * NBUF

        for c in range(min(NBUF, n_chunks)):
            read_h[c] = pltpu.make_async_copy(
                w_any.at[pl.ds(c * BS, BS)], bufs[c], rsems[c])
            read_h[c].start()
        for c in range(n_chunks):
            s = c % NBUF
            read_h[s].wait()
            write_h[s] = []
            for b in range(batch):
                h = pltpu.make_async_copy(
                    bufs[s], o_any.at[b, pl.ds(c * BS, BS)], wsems[s])
                h.start()
                write_h[s].append(h)
            nxt = c + NBUF
            if nxt < n_chunks:
                for h in write_h[s]:
                    h.wait()
                write_h[s] = None
                read_h[s] = pltpu.make_async_copy(
                    w_any.at[pl.ds(nxt * BS, BS)], bufs[s], rsems[s])
                read_h[s].start()
        for s in range(NBUF):
            if write_h[s] is not None:
                for h in write_h[s]:
                    h.wait()

    return pl.pallas_call(
        body,
        in_specs=[pl.BlockSpec(memory_space=pl.ANY)],
        out_specs=pl.BlockSpec(memory_space=pl.ANY),
        out_shape=jax.ShapeDtypeStruct((batch, seq_len, d_model), W_pos.dtype),
        scratch_shapes=(
            [pltpu.VMEM((BS, d_model), jnp.float32)] * NBUF
            + [pltpu.SemaphoreType.DMA] * (2 * NBUF)
        ),
    )(W_pos)
